# baseline (speedup 1.0000x reference)
import functools

import jax
import jax.numpy as jnp
from jax import lax
from jax.experimental import pallas as pl
from jax.experimental.pallas import tpu as pltpu
from jax.experimental.pallas import tpu_sc as plsc

NQ = 1024
D = 32
L = 100000
K = 8
KB = 32
NBLK = L // KB
TILE = 4000
NT = L // TILE
BPT = TILE // KB
NQC = 256
RC = 128
NW = 32

_NEG = float("-inf")
_IMAX = 2147483647


def _stage_a(q_ref, wqt_ref, bq_ref, keys_ref, qp_out, bm_out, qp_sc):
    t = pl.program_id(0)

    @pl.when(t == 0)
    def _():
        qp = jnp.dot(q_ref[...], wqt_ref[...],
                     preferred_element_type=jnp.float32) + bq_ref[...]
        qp_sc[...] = qp
        qp_out[...] = qp

    scores = lax.dot_general(keys_ref[...], qp_sc[...],
                             (((1,), (1,)), ((), ())),
                             preferred_element_type=jnp.float32)
    bm_out[...] = jnp.max(scores.reshape(BPT, KB, NQ), axis=1)[None]


def _stage_b(bm_ref, ids_ref):
    bm = bm_ref[...]
    biota = lax.broadcasted_iota(jnp.int32, (NBLK, NQC), 0)
    ids = []
    for _ in range(K):
        m = jnp.max(bm, axis=0)
        sel = jnp.where(bm == m[None, :], biota, jnp.int32(NBLK))
        idx = jnp.min(sel, axis=0)
        ids.append(idx)
        bm = jnp.where(biota == idx[None, :], _NEG, bm)
    ids_ref[...] = jnp.stack(ids, axis=0)


def _stage_c(gk_ref, qpe_ref, bid_ref, w_ref, ti_ref):
    G = gk_ref[...]
    qpe = qpe_ref[...]
    di = lax.broadcasted_iota(jnp.int32, (D, KB * D), 0)
    ci = lax.broadcasted_iota(jnp.int32, (D, KB * D), 1)
    E = (ci % D == di).astype(jnp.float32)
    prod = G * jnp.dot(qpe, E, preferred_element_type=jnp.float32)
    ui = lax.broadcasted_iota(jnp.int32, (KB * D, KB), 0)
    uj = lax.broadcasted_iota(jnp.int32, (KB * D, KB), 1)
    M = (ui // D == uj).astype(jnp.float32)
    cand = jnp.dot(prod, M,
                   preferred_element_type=jnp.float32).reshape(RC, K, KB)
    bid = bid_ref[...]
    gidx = bid[:, :, None] * KB + lax.broadcasted_iota(
        jnp.int32, (RC, K, KB), 2)
    vals, idxs = [], []
    for _ in range(K):
        m = jnp.max(cand, axis=(1, 2), keepdims=True)
        sel = jnp.where(cand == m, gidx, _IMAX)
        ix = jnp.min(sel, axis=(1, 2), keepdims=True)
        vals.append(m[:, 0])
        idxs.append(ix[:, 0])
        cand = jnp.where(gidx == ix, _NEG, cand)
    v = jnp.concatenate(vals, axis=1)
    e = jnp.exp(v - v[:, 0:1])
    w_ref[...] = e / jnp.sum(e, axis=1, keepdims=True)
    ti_ref[...] = jnp.concatenate(idxs, axis=1)


def _stage_d(gv_ref, w_ref, ti_ref, q_ref, wot_ref, bo_ref, out_ref):
    gv = gv_ref[...]
    w = w_ref[...]
    sub = ti_ref[...] % 4
    sel = (lax.broadcasted_iota(jnp.int32, (NQ, K, 4), 2)
           == sub[:, :, None]).astype(jnp.float32)
    mo = jnp.sum(gv * (sel * w[:, :, None])[..., None], axis=(1, 2))
    out_ref[...] = (jnp.dot(mo, wot_ref[...],
                            preferred_element_type=jnp.float32)
                    + bo_ref[...] + q_ref[...])


def _make_sc_gather(n_rows, d_row, n_idx, chunk):
    per_w = n_idx // NW
    n_chunks = per_w // chunk
    mesh = plsc.VectorSubcoreMesh(core_axis_name="c", subcore_axis_name="s")

    @functools.partial(
        pl.kernel,
        out_type=jax.ShapeDtypeStruct((n_idx, d_row), jnp.float32),
        mesh=mesh,
        scratch_types=[
            pltpu.VMEM((chunk,), jnp.int32),
            pltpu.VMEM((chunk, d_row), jnp.float32),
            pltpu.SemaphoreType.DMA,
        ],
    )
    def gather(table_hbm, idx_hbm, out_hbm, idx_v, rows_v, sem):
        wid = lax.axis_index("s") * 2 + lax.axis_index("c")
        for c in range(n_chunks):
            base = wid * per_w + c * chunk
            pltpu.sync_copy(idx_hbm.at[pl.ds(base, chunk)], idx_v)
            pltpu.async_copy(table_hbm.at[idx_v], rows_v, sem).wait()
            pltpu.sync_copy(rows_v, out_hbm.at[pl.ds(base, chunk)])

    return gather


def kernel(query, memory_keys, memory_values, Wq, bq, Wo, bo):
    b, s, _ = query.shape
    qf = query.reshape(NQ, D)
    key_blocks = memory_keys.reshape(NBLK, KB * D)

    qp, bm = pl.pallas_call(
        _stage_a,
        grid=(NT,),
        in_specs=[
            pl.BlockSpec((NQ, D), lambda t: (0, 0)),
            pl.BlockSpec((D, D), lambda t: (0, 0)),
            pl.BlockSpec((1, D), lambda t: (0, 0)),
            pl.BlockSpec((TILE, D), lambda t: (t, 0)),
        ],
        out_specs=[
            pl.BlockSpec((NQ, D), lambda t: (0, 0)),
            pl.BlockSpec((1, BPT, NQ), lambda t: (t, 0, 0)),
        ],
        out_shape=[
            jax.ShapeDtypeStruct((NQ, D), jnp.float32),
            jax.ShapeDtypeStruct((NT, BPT, NQ), jnp.float32),
        ],
        scratch_shapes=[pltpu.VMEM((NQ, D), jnp.float32)],
    )(qf, Wq.T, bq.reshape(1, D), memory_keys)

    ids = pl.pallas_call(
        _stage_b,
        grid=(NQ // NQC,),
        in_specs=[pl.BlockSpec((NBLK, NQC), lambda c: (0, c))],
        out_specs=pl.BlockSpec((K, NQC), lambda c: (0, c)),
        out_shape=jax.ShapeDtypeStruct((K, NQ), jnp.int32),
    )(bm.reshape(NBLK, NQ))

    bid = ids.T
    gk = _make_sc_gather(NBLK, KB * D, NQ * K, 64)(
        key_blocks, bid.reshape(NQ * K))

    w, ti = pl.pallas_call(
        _stage_c,
        grid=(NQ // RC,),
        in_specs=[
            pl.BlockSpec((RC * K, KB * D), lambda c: (c, 0)),
            pl.BlockSpec((RC * K, D), lambda c: (c, 0)),
            pl.BlockSpec((RC, K), lambda c: (c, 0)),
        ],
        out_specs=[
            pl.BlockSpec((RC, K), lambda c: (c, 0)),
            pl.BlockSpec((RC, K), lambda c: (c, 0)),
        ],
        out_shape=[
            jax.ShapeDtypeStruct((NQ, K), jnp.float32),
            jax.ShapeDtypeStruct((NQ, K), jnp.int32),
        ],
    )(gk, jnp.repeat(qp, K, axis=0), bid)

    gv = _make_sc_gather(L // 4, 4 * D, NQ * K, 64)(
        memory_values.reshape(L // 4, 4 * D), (ti // 4).reshape(NQ * K))

    out = pl.pallas_call(
        _stage_d,
        in_specs=[
            pl.BlockSpec((NQ, K, 4, D), lambda: (0, 0, 0, 0)),
            pl.BlockSpec((NQ, K), lambda: (0, 0)),
            pl.BlockSpec((NQ, K), lambda: (0, 0)),
            pl.BlockSpec((NQ, D), lambda: (0, 0)),
            pl.BlockSpec((D, D), lambda: (0, 0)),
            pl.BlockSpec((1, D), lambda: (0, 0)),
        ],
        out_specs=pl.BlockSpec((NQ, D), lambda: (0, 0)),
        out_shape=jax.ShapeDtypeStruct((NQ, D), jnp.float32),
    )(gv.reshape(NQ, K, 4, D), w, ti, qf, Wo.T, bo.reshape(1, D))

    return out.reshape(b, s, D)


# speedup vs baseline: 1.0233x; 1.0233x over previous
import functools

import jax
import jax.numpy as jnp
from jax import lax
from jax.experimental import pallas as pl
from jax.experimental.pallas import tpu as pltpu
from jax.experimental.pallas import tpu_sc as plsc

NQ = 1024
D = 32
L = 100000
K = 8
KB = 32
NBLK = L // KB
TILE = 4000
NT = L // TILE
BPT = TILE // KB
NQC = 256
RC = 128
NW = 32

_NEG = float("-inf")
_IMAX = 2147483647


def _stage_a(q_ref, wqt_ref, bq_ref, keys_ref, qp_out, ids_out, qp_sc, bm_sc):
    t = pl.program_id(0)

    @pl.when(t == 0)
    def _():
        qp = jnp.dot(q_ref[...], wqt_ref[...],
                     preferred_element_type=jnp.float32) + bq_ref[...]
        qp_sc[...] = qp
        qp_out[...] = qp

    scores = lax.dot_general(keys_ref[...], qp_sc[...],
                             (((1,), (1,)), ((), ())),
                             preferred_element_type=jnp.float32)
    bm_sc[t] = jnp.max(scores.reshape(BPT, KB, NQ), axis=1)

    @pl.when(t == NT - 1)
    def _():
        for c in range(NQ // NQC):
            bm = bm_sc[:, :, c * NQC:(c + 1) * NQC]
            i0 = lax.broadcasted_iota(jnp.int32, (NT, BPT, NQC), 0)
            i1 = lax.broadcasted_iota(jnp.int32, (NT, BPT, NQC), 1)
            biota = i0 * BPT + i1
            ids = []
            for _ in range(K):
                m = jnp.max(bm, axis=(0, 1))
                sel = jnp.where(bm == m[None, None, :], biota, jnp.int32(NBLK))
                idx = jnp.min(sel, axis=(0, 1))
                ids.append(idx)
                bm = jnp.where(biota == idx[None, None, :], _NEG, bm)
            ids_out[:, c * NQC:(c + 1) * NQC] = jnp.stack(ids, axis=0)


def _stage_c(gk_ref, qpe_ref, bid_ref, w_ref, ti_ref):
    G = gk_ref[...]
    qpe = qpe_ref[...]
    di = lax.broadcasted_iota(jnp.int32, (D, KB * D), 0)
    ci = lax.broadcasted_iota(jnp.int32, (D, KB * D), 1)
    E = (ci % D == di).astype(jnp.float32)
    prod = G * jnp.dot(qpe, E, preferred_element_type=jnp.float32)
    ui = lax.broadcasted_iota(jnp.int32, (KB * D, KB), 0)
    uj = lax.broadcasted_iota(jnp.int32, (KB * D, KB), 1)
    M = (ui // D == uj).astype(jnp.float32)
    cand = jnp.dot(prod, M,
                   preferred_element_type=jnp.float32).reshape(RC, K, KB)
    bid = bid_ref[...]
    gidx = bid[:, :, None] * KB + lax.broadcasted_iota(
        jnp.int32, (RC, K, KB), 2)
    vals, idxs = [], []
    for _ in range(K):
        m = jnp.max(cand, axis=(1, 2), keepdims=True)
        sel = jnp.where(cand == m, gidx, _IMAX)
        ix = jnp.min(sel, axis=(1, 2), keepdims=True)
        vals.append(m[:, 0])
        idxs.append(ix[:, 0])
        cand = jnp.where(gidx == ix, _NEG, cand)
    v = jnp.concatenate(vals, axis=1)
    e = jnp.exp(v - v[:, 0:1])
    w_ref[...] = e / jnp.sum(e, axis=1, keepdims=True)
    ti_ref[...] = jnp.concatenate(idxs, axis=1)


def _stage_d(gv_ref, w_ref, ti_ref, q_ref, wot_ref, bo_ref, out_ref):
    gv = gv_ref[...]
    w = w_ref[...]
    sub = ti_ref[...] % 4
    sel = (lax.broadcasted_iota(jnp.int32, (NQ, K, 4), 2)
           == sub[:, :, None]).astype(jnp.float32)
    mo = jnp.sum(gv * (sel * w[:, :, None])[..., None], axis=(1, 2))
    out_ref[...] = (jnp.dot(mo, wot_ref[...],
                            preferred_element_type=jnp.float32)
                    + bo_ref[...] + q_ref[...])


def _make_sc_gather(n_rows, d_row, n_idx, chunk):
    per_w = n_idx // NW
    n_chunks = per_w // chunk
    mesh = plsc.VectorSubcoreMesh(core_axis_name="c", subcore_axis_name="s")

    @functools.partial(
        pl.kernel,
        out_type=jax.ShapeDtypeStruct((n_idx, d_row), jnp.float32),
        mesh=mesh,
        scratch_types=[
            pltpu.VMEM((chunk,), jnp.int32),
            pltpu.VMEM((chunk, d_row), jnp.float32),
            pltpu.SemaphoreType.DMA,
        ],
    )
    def gather(table_hbm, idx_hbm, out_hbm, idx_v, rows_v, sem):
        wid = lax.axis_index("s") * 2 + lax.axis_index("c")
        for c in range(n_chunks):
            base = wid * per_w + c * chunk
            pltpu.sync_copy(idx_hbm.at[pl.ds(base, chunk)], idx_v)
            pltpu.async_copy(table_hbm.at[idx_v], rows_v, sem).wait()
            pltpu.sync_copy(rows_v, out_hbm.at[pl.ds(base, chunk)])

    return gather


def kernel(query, memory_keys, memory_values, Wq, bq, Wo, bo):
    b, s, _ = query.shape
    qf = query.reshape(NQ, D)
    key_blocks = memory_keys.reshape(NBLK, KB * D)

    qp, ids = pl.pallas_call(
        _stage_a,
        grid=(NT,),
        in_specs=[
            pl.BlockSpec((NQ, D), lambda t: (0, 0)),
            pl.BlockSpec((D, D), lambda t: (0, 0)),
            pl.BlockSpec((1, D), lambda t: (0, 0)),
            pl.BlockSpec((TILE, D), lambda t: (t, 0)),
        ],
        out_specs=[
            pl.BlockSpec((NQ, D), lambda t: (0, 0)),
            pl.BlockSpec((K, NQ), lambda t: (0, 0)),
        ],
        out_shape=[
            jax.ShapeDtypeStruct((NQ, D), jnp.float32),
            jax.ShapeDtypeStruct((K, NQ), jnp.int32),
        ],
        scratch_shapes=[pltpu.VMEM((NQ, D), jnp.float32),
                        pltpu.VMEM((NT, BPT, NQ), jnp.float32)],
    )(qf, Wq.T, bq.reshape(1, D), memory_keys)

    bid = ids.T
    gk = _make_sc_gather(NBLK, KB * D, NQ * K, 64)(
        key_blocks, bid.reshape(NQ * K))

    w, ti = pl.pallas_call(
        _stage_c,
        grid=(NQ // RC,),
        in_specs=[
            pl.BlockSpec((RC * K, KB * D), lambda c: (c, 0)),
            pl.BlockSpec((RC * K, D), lambda c: (c, 0)),
            pl.BlockSpec((RC, K), lambda c: (c, 0)),
        ],
        out_specs=[
            pl.BlockSpec((RC, K), lambda c: (c, 0)),
            pl.BlockSpec((RC, K), lambda c: (c, 0)),
        ],
        out_shape=[
            jax.ShapeDtypeStruct((NQ, K), jnp.float32),
            jax.ShapeDtypeStruct((NQ, K), jnp.int32),
        ],
    )(gk, jnp.repeat(qp, K, axis=0), bid)

    gv = _make_sc_gather(L // 4, 4 * D, NQ * K, 64)(
        memory_values.reshape(L // 4, 4 * D), (ti // 4).reshape(NQ * K))

    out = pl.pallas_call(
        _stage_d,
        in_specs=[
            pl.BlockSpec((NQ, K, 4, D), lambda: (0, 0, 0, 0)),
            pl.BlockSpec((NQ, K), lambda: (0, 0)),
            pl.BlockSpec((NQ, K), lambda: (0, 0)),
            pl.BlockSpec((NQ, D), lambda: (0, 0)),
            pl.BlockSpec((D, D), lambda: (0, 0)),
            pl.BlockSpec((1, D), lambda: (0, 0)),
        ],
        out_specs=pl.BlockSpec((NQ, D), lambda: (0, 0)),
        out_shape=jax.ShapeDtypeStruct((NQ, D), jnp.float32),
    )(gv.reshape(NQ, K, 4, D), w, ti, qf, Wo.T, bo.reshape(1, D))

    return out.reshape(b, s, D)
